# baseline (speedup 1.0000x reference)
import functools

import jax
import jax.numpy as jnp
from jax import lax
from jax.experimental import pallas as pl
from jax.experimental.pallas import tpu as pltpu
from jax.experimental.pallas import tpu_sc as plsc

L = 16
CH = 16
H = 8


def _make_sc_gather(n_nodes, n_graphs, d_in, chunk, n_workers, search_iters):
  nc = 2
  mesh = plsc.VectorSubcoreMesh(core_axis_name="c", subcore_axis_name="s",
                                num_cores=nc)
  last = n_workers - 1
  avail = n_nodes - (last * chunk - H)
  fill = chunk + 2 * H - avail

  @functools.partial(
      pl.kernel,
      mesh=mesh,
      compiler_params=pltpu.CompilerParams(needs_layout_passes=False),
      out_type=jax.ShapeDtypeStruct((n_graphs, d_in), jnp.float32),
      scratch_types=[
          pltpu.VMEM((chunk + 2 * H + L,), jnp.int32),
          pltpu.VMEM((n_graphs,), jnp.int32),
          pltpu.VMEM((1, CH), jnp.int32),
          pltpu.VMEM((1, CH), jnp.int32),
          pltpu.VMEM((1, CH), jnp.int32),
          pltpu.VMEM((1, CH), jnp.int32),
          pltpu.VMEM((CH, d_in), jnp.float32),
          pltpu.VMEM((CH, d_in), jnp.float32),
          pltpu.SemaphoreType.DMA,
          pltpu.SemaphoreType.DMA,
          pltpu.SemaphoreType.DMA,
      ],
  )
  def sc_gather(batch, ego, gnn_x, dsx_out, chunk_v, ego_v, pos0, gid0, pos1,
                gid1, rows0, rows1, sem_c, sem_g, sem_s):
    wid = lax.axis_index("s") * nc + lax.axis_index("c")
    base = wid * chunk

    cp_ego = pltpu.async_copy(ego, ego_v, sem_c)

    @pl.when(wid == 0)
    def _():
      pltpu.async_copy(batch.at[pl.ds(0, chunk + H)],
                       chunk_v.at[pl.ds(H, chunk + H)], sem_g).wait()

    @pl.when(jnp.logical_and(wid > 0, wid < last))
    def _():
      pltpu.async_copy(batch.at[pl.ds(base - H, chunk + 2 * H)],
                       chunk_v.at[pl.ds(0, chunk + 2 * H)], sem_g).wait()

    @pl.when(wid == last)
    def _():
      pltpu.async_copy(batch.at[pl.ds(last * chunk - H, avail)],
                       chunk_v.at[pl.ds(0, avail)], sem_g).wait()
      rep = (jnp.zeros((L,), jnp.int32)
             + chunk_v[pl.ds(avail - L, L)][L - 1])

      def fstep(k, carry):
        chunk_v[pl.ds(avail + k * L, L)] = rep
        return carry

      lax.fori_loop(0, -(-fill // L), fstep, jnp.int32(0))

    cp_ego.wait()

    g_lo = chunk_v[pl.ds(0, L)][H - 1]
    g_lo = jnp.where(wid == 0, -1, g_lo)
    g_hi = chunk_v[pl.ds(chunk, L)][H - 1]
    gcnt = g_hi - g_lo

    def build_idx(j, posr, gidr):
      first = g_lo + 1 + j * CH
      for k in range(CH // L):
        g = jnp.minimum(first + k * L + lax.iota(jnp.int32, L), g_hi)
        def sstep(_, carry):
          lo, hi = carry
          mid = (lo + hi) >> 1
          less = plsc.load_gather(chunk_v, [mid]) < g
          return jnp.where(less, mid + 1, lo), jnp.where(less, hi, mid)

        lo, hi = lax.fori_loop(
            0, search_iters,
            sstep, (jnp.zeros((L,), jnp.int32) + H,
                    jnp.zeros((L,), jnp.int32) + (chunk + H - 1)))
        pos = base + lo - H + plsc.load_gather(ego_v, [g])
        posr[0, pl.ds(k * L, L)] = pos
        gidr[0, pl.ds(k * L, L)] = g

    @pl.when(gcnt > 0)
    def _():
      n_slabs = (gcnt + CH - 1) // CH
      build_idx(0, pos0, gid0)
      pltpu.async_copy(gnn_x.at[pos0.at[0]], rows0, sem_g)

      def step(j, cur_pos, cur_gid, cur_rows, nxt_pos, nxt_gid, nxt_rows):
        pltpu.make_async_copy(gnn_x.at[cur_pos.at[0]], cur_rows, sem_g).wait()

        @pl.when(j + 1 < n_slabs)
        def _():
          build_idx(j + 1, nxt_pos, nxt_gid)
          pltpu.async_copy(gnn_x.at[nxt_pos.at[0]], nxt_rows, sem_g)

        pltpu.async_copy(cur_rows, dsx_out.at[cur_gid.at[0]], sem_s).wait()

      def slab(j, carry):
        @pl.when(j % 2 == 0)
        def _():
          step(j, pos0, gid0, rows0, pos1, gid1, rows1)

        @pl.when(j % 2 == 1)
        def _():
          step(j, pos1, gid1, rows1, pos0, gid0, rows0)

        return carry

      lax.fori_loop(0, n_slabs, slab, jnp.int32(0))

  return sc_gather


def _tc_route_body(n_experts, out_ch, f1, dsx_ref, wc_ref, wo_ref, out_ref):
  ds = dsx_ref[...]
  scores = lax.dot_general(ds, wc_ref[...], (((1,), (1,)), ((), ())),
                           preferred_element_type=jnp.float32)
  mx = jnp.max(scores, axis=1, keepdims=True)
  col = lax.broadcasted_iota(jnp.int32, scores.shape, 1)
  cidx = jnp.min(jnp.where(scores == mx, col, n_experts), axis=1,
                 keepdims=True)
  p_all = lax.dot_general(ds, wo_ref[...], (((1,), (1,)), ((), ())),
                          preferred_element_type=jnp.float32)
  grp = lax.broadcasted_iota(jnp.int32, p_all.shape, 1) // out_ch
  masked = jnp.where(grp == cidx, p_all, 0.0)
  f2 = n_experts // f1
  w1 = f2 * out_ch
  t = masked[:, 0:w1]
  for s2 in range(1, f1):
    t = t + masked[:, s2 * w1:(s2 + 1) * w1]
  acc = t[:, 0:out_ch]
  for e in range(1, f2):
    acc = acc + t[:, e * out_ch:(e + 1) * out_ch]
  out_ref[...] = acc


def kernel(gnn_x, batch, ego_idx, W_center, W_out):
  n_nodes, d_in = gnn_x.shape
  n_graphs = ego_idx.shape[0]
  n_experts, out_ch, _ = W_out.shape

  batch = batch.astype(jnp.int32)
  ego = ego_idx.astype(jnp.int32)

  n_workers = 32
  chunk = -(-n_nodes // (n_workers * L)) * L
  search_iters = max(1, (chunk - 1).bit_length())

  sc_gather = _make_sc_gather(n_nodes, n_graphs, d_in, chunk, n_workers,
                              search_iters)
  ds_x = sc_gather(batch, ego, gnn_x)

  wo_flat = W_out.reshape(n_experts * out_ch, d_in)
  f1 = max(d for d in range(1, int(n_experts**0.5) + 1) if n_experts % d == 0)
  bt = 512
  grid = n_graphs // bt
  logits = pl.pallas_call(
      functools.partial(_tc_route_body, n_experts, out_ch, f1),
      grid=(grid,),
      in_specs=[
          pl.BlockSpec((bt, d_in), lambda i: (i, 0)),
          pl.BlockSpec((n_experts, d_in), lambda i: (0, 0)),
          pl.BlockSpec((n_experts * out_ch, d_in), lambda i: (0, 0)),
      ],
      out_specs=pl.BlockSpec((bt, out_ch), lambda i: (i, 0)),
      out_shape=jax.ShapeDtypeStruct((n_graphs, out_ch), jnp.float32),
  )(ds_x, W_center, wo_flat)
  return logits


# speedup vs baseline: 1.0327x; 1.0327x over previous
import functools

import jax
import jax.numpy as jnp
from jax import lax
from jax.experimental import pallas as pl
from jax.experimental.pallas import tpu as pltpu
from jax.experimental.pallas import tpu_sc as plsc

L = 16
CH = 16
H = 8


def _make_sc_gather(n_nodes, n_graphs, d_in, chunk, n_workers, search_iters):
  nc = 2
  mesh = plsc.VectorSubcoreMesh(core_axis_name="c", subcore_axis_name="s",
                                num_cores=nc)
  last = n_workers - 1
  avail = n_nodes - (last * chunk - H)
  fill = chunk + 2 * H - avail

  @functools.partial(
      pl.kernel,
      mesh=mesh,
      compiler_params=pltpu.CompilerParams(needs_layout_passes=False),
      out_type=jax.ShapeDtypeStruct((n_graphs, d_in), jnp.float32),
      scratch_types=[
          pltpu.VMEM((chunk + 2 * H + L,), jnp.int32),
          pltpu.VMEM((n_graphs,), jnp.int32),
          pltpu.VMEM((1, CH), jnp.int32),
          pltpu.VMEM((1, CH), jnp.int32),
          pltpu.VMEM((1, CH), jnp.int32),
          pltpu.VMEM((1, CH), jnp.int32),
          pltpu.VMEM((CH, d_in), jnp.float32),
          pltpu.VMEM((CH, d_in), jnp.float32),
          pltpu.SemaphoreType.DMA,
          pltpu.SemaphoreType.DMA,
          pltpu.SemaphoreType.DMA,
      ],
  )
  def sc_gather(batch, ego, gnn_x, dsx_out, chunk_v, ego_v, pos0, gid0, pos1,
                gid1, rows0, rows1, sem_c, sem_g, sem_s):
    wid = lax.axis_index("s") * nc + lax.axis_index("c")
    base = wid * chunk

    cp_ego = pltpu.async_copy(ego, ego_v, sem_c)

    @pl.when(wid == 0)
    def _():
      pltpu.async_copy(batch.at[pl.ds(0, chunk + H)],
                       chunk_v.at[pl.ds(H, chunk + H)], sem_g).wait()

    @pl.when(jnp.logical_and(wid > 0, wid < last))
    def _():
      pltpu.async_copy(batch.at[pl.ds(base - H, chunk + 2 * H)],
                       chunk_v.at[pl.ds(0, chunk + 2 * H)], sem_g).wait()

    @pl.when(wid == last)
    def _():
      pltpu.async_copy(batch.at[pl.ds(last * chunk - H, avail)],
                       chunk_v.at[pl.ds(0, avail)], sem_g).wait()
      rep = (jnp.zeros((L,), jnp.int32)
             + chunk_v[pl.ds(avail - L, L)][L - 1])
      for k in range(-(-fill // L)):
        chunk_v[pl.ds(avail + k * L, L)] = rep

    cp_ego.wait()

    g_lo = chunk_v[pl.ds(0, L)][H - 1]
    g_lo = jnp.where(wid == 0, -1, g_lo)
    g_hi = chunk_v[pl.ds(chunk, L)][H - 1]
    gcnt = g_hi - g_lo

    def build_idx(j, posr, gidr):
      first = g_lo + 1 + j * CH
      for k in range(CH // L):
        g = jnp.minimum(first + k * L + lax.iota(jnp.int32, L), g_hi)
        lo = jnp.zeros((L,), jnp.int32) + H
        hi = jnp.zeros((L,), jnp.int32) + (chunk + H - 1)
        for _ in range(search_iters):
          mid = (lo + hi) >> 1
          less = plsc.load_gather(chunk_v, [mid]) < g
          lo = jnp.where(less, mid + 1, lo)
          hi = jnp.where(less, hi, mid)
        pos = base + lo - H + plsc.load_gather(ego_v, [g])
        posr[0, pl.ds(k * L, L)] = pos
        gidr[0, pl.ds(k * L, L)] = g

    @pl.when(gcnt > 0)
    def _():
      n_slabs = (gcnt + CH - 1) // CH
      build_idx(0, pos0, gid0)
      pltpu.async_copy(gnn_x.at[pos0.at[0]], rows0, sem_g)

      def step(j, cur_pos, cur_gid, cur_rows, nxt_pos, nxt_gid, nxt_rows):
        pltpu.make_async_copy(gnn_x.at[cur_pos.at[0]], cur_rows, sem_g).wait()

        @pl.when(j + 1 < n_slabs)
        def _():
          build_idx(j + 1, nxt_pos, nxt_gid)
          pltpu.async_copy(gnn_x.at[nxt_pos.at[0]], nxt_rows, sem_g)

        pltpu.async_copy(cur_rows, dsx_out.at[cur_gid.at[0]], sem_s).wait()

      def slab(j, carry):
        @pl.when(j % 2 == 0)
        def _():
          step(j, pos0, gid0, rows0, pos1, gid1, rows1)

        @pl.when(j % 2 == 1)
        def _():
          step(j, pos1, gid1, rows1, pos0, gid0, rows0)

        return carry

      lax.fori_loop(0, n_slabs, slab, jnp.int32(0))

  return sc_gather


def _tc_route_body(n_experts, out_ch, f1, dsx_ref, wc_ref, wo_ref, out_ref):
  ds = dsx_ref[...]
  scores = lax.dot_general(ds, wc_ref[...], (((1,), (1,)), ((), ())),
                           preferred_element_type=jnp.float32)
  mx = jnp.max(scores, axis=1, keepdims=True)
  col = lax.broadcasted_iota(jnp.int32, scores.shape, 1)
  cidx = jnp.min(jnp.where(scores == mx, col, n_experts), axis=1,
                 keepdims=True)
  p_all = lax.dot_general(ds, wo_ref[...], (((1,), (1,)), ((), ())),
                          preferred_element_type=jnp.float32)
  grp = lax.broadcasted_iota(jnp.int32, p_all.shape, 1) // out_ch
  masked = jnp.where(grp == cidx, p_all, 0.0)
  f2 = n_experts // f1
  w1 = f2 * out_ch
  t = masked[:, 0:w1]
  for s2 in range(1, f1):
    t = t + masked[:, s2 * w1:(s2 + 1) * w1]
  acc = t[:, 0:out_ch]
  for e in range(1, f2):
    acc = acc + t[:, e * out_ch:(e + 1) * out_ch]
  out_ref[...] = acc


def kernel(gnn_x, batch, ego_idx, W_center, W_out):
  n_nodes, d_in = gnn_x.shape
  n_graphs = ego_idx.shape[0]
  n_experts, out_ch, _ = W_out.shape

  batch = batch.astype(jnp.int32)
  ego = ego_idx.astype(jnp.int32)

  n_workers = 32
  chunk = -(-n_nodes // (n_workers * L)) * L
  search_iters = max(1, (chunk - 1).bit_length())

  sc_gather = _make_sc_gather(n_nodes, n_graphs, d_in, chunk, n_workers,
                              search_iters)
  ds_x = sc_gather(batch, ego, gnn_x)

  wo_flat = W_out.reshape(n_experts * out_ch, d_in)
  f1 = max(d for d in range(1, int(n_experts**0.5) + 1) if n_experts % d == 0)
  bt = 1024
  grid = n_graphs // bt
  logits = pl.pallas_call(
      functools.partial(_tc_route_body, n_experts, out_ch, f1),
      grid=(grid,),
      in_specs=[
          pl.BlockSpec((bt, d_in), lambda i: (i, 0)),
          pl.BlockSpec((n_experts, d_in), lambda i: (0, 0)),
          pl.BlockSpec((n_experts * out_ch, d_in), lambda i: (0, 0)),
      ],
      out_specs=pl.BlockSpec((bt, out_ch), lambda i: (i, 0)),
      out_shape=jax.ShapeDtypeStruct((n_graphs, out_ch), jnp.float32),
  )(ds_x, W_center, wo_flat)
  return logits
